# baseline (speedup 1.0000x reference)
import functools

import jax
import jax.numpy as jnp
from jax import lax
from jax.experimental import pallas as pl
from jax.experimental.pallas import tpu as pltpu
from jax.experimental.pallas import tpu_sc as plsc

_L = 128
_NW = 32
_C = 16


def _ceil_to(x, m):
  return -(-x // m) * m


def _make_sc_gather_add(Ep, NP):
  GS = 64
  GPW = Ep // (GS * _NW)
  rows_pt = NP // 16
  mesh = plsc.VectorSubcoreMesh(core_axis_name="c", subcore_axis_name="s")
  assert GPW % 16 == 0 and GPW >= 32

  @functools.partial(
      pl.kernel,
      out_type=jax.ShapeDtypeStruct((2 * NP, _C), jnp.float32),
      mesh=mesh,
      compiler_params=pltpu.CompilerParams(use_tc_tiling_on_sc=False),
      scratch_types=[
          pltpu.VMEM((GPW, GS), jnp.int32),
          pltpu.VMEM((GPW, GS), jnp.int32),
          [pltpu.VMEM((GS, _C), jnp.float32) for _ in range(16)],
          pltpu.VMEM((_L, _C), jnp.float32),
          pltpu.VMEM_SHARED((NP, _C), jnp.float32),
          [pltpu.SemaphoreType.DMA for _ in range(16)],
      ],
  )
  def conv(tab_h, gidx2_h, dst2_h, out_h, gv, dv, rows, zb, acc, sem):
    c = lax.axis_index("c")
    s = lax.axis_index("s")
    w = s * 2 + c

    def zf(i, _):
      zb[i, :] = jnp.zeros((_C,), jnp.float32)
      return 0
    lax.fori_loop(0, _L, zf, 0)

    base = pl.multiple_of(s * rows_pt, _L)
    def zc(i, _):
      pltpu.sync_copy(zb, acc.at[pl.ds(pl.multiple_of(base + i * _L, _L), _L)])
      return 0
    lax.fori_loop(0, rows_pt // _L, zc, 0)

    g0 = pl.multiple_of(w * GPW, 16)
    pltpu.sync_copy(gidx2_h.at[pl.ds(g0, GPW)], gv)
    pltpu.sync_copy(dst2_h.at[pl.ds(g0, GPW)], dv)
    plsc.subcore_barrier()

    def gather(j, b):
      pltpu.async_copy(tab_h.at[gv.at[j]], rows[b], sem[b])

    def gwait(b):
      pltpu.make_async_copy(tab_h.at[gv.at[0]], rows[b], sem[b]).wait()

    def scatter(j, b):
      pltpu.async_copy(rows[b], acc.at[dv.at[j]], sem[b], add=True)

    def swait(b):
      pltpu.make_async_copy(rows[b], acc.at[dv.at[0]], sem[b]).wait()

    for i in range(8):
      gather(i, i)
    for i in range(8):
      gather(i + 8, i + 8)
      gwait(i)
      scatter(i, i)

    def body16(o, _):
      jb = 8 + o * 16
      for i in range(16):
        j = jb + i
        bj = (8 + i) % 16
        bn = i
        swait(bn)
        gather(j + 8, bn)
        gwait(bj)
        scatter(j, bj)
      return 0
    lax.fori_loop(0, (GPW - 16) // 16, body16, 0)

    for i in range(8):
      gwait(8 + i)
      scatter(GPW - 8 + i, 8 + i)
    for b in range(16):
      swait(b)

    plsc.subcore_barrier()
    pltpu.sync_copy(acc.at[pl.ds(base, rows_pt)],
                    out_h.at[pl.ds(pl.multiple_of(c * NP + base, _L), rows_pt)])

  return conv


def _make_sc_conv_in(Ep, NP, K, nstride):
  GPW = Ep // (_L * _NW)
  EPW = Ep // _NW
  rows_pt = NP // 16
  mesh = plsc.VectorSubcoreMesh(core_axis_name="c", subcore_axis_name="s")

  @functools.partial(
      pl.kernel,
      out_type=(jax.ShapeDtypeStruct((Ep // _L, _L), jnp.int32),
                jax.ShapeDtypeStruct((2 * NP, _C), jnp.float32)),
      mesh=mesh,
      compiler_params=pltpu.CompilerParams(use_tc_tiling_on_sc=False),
      scratch_types=[
          pltpu.VMEM((8 * _L,), jnp.int32),
          pltpu.VMEM((8 * _L,), jnp.int32),
          pltpu.VMEM((GPW, _L), jnp.int32),
          pltpu.VMEM((GPW, _L), jnp.int32),
          [pltpu.VMEM((_L, _C), jnp.float32) for _ in range(8)],
          pltpu.VMEM((_L, _C), jnp.float32),
          pltpu.VMEM_SHARED((NP, _C), jnp.float32),
          [pltpu.SemaphoreType.DMA for _ in range(8)],
      ],
  )
  def convin(tab_h, src_h, kidx_h, dst2_h, gidx2_h, out_h, srcb, kb, gv, dv,
             rows, zb, acc, sem):
    c = lax.axis_index("c")
    s = lax.axis_index("s")
    w = s * 2 + c

    def zf(i, _):
      zb[i, :] = jnp.zeros((_C,), jnp.float32)
      return 0
    lax.fori_loop(0, _L, zf, 0)
    base = pl.multiple_of(s * rows_pt, _L)
    def zc(i, _):
      pltpu.sync_copy(zb, acc.at[pl.ds(pl.multiple_of(base + i * _L, _L), _L)])
      return 0
    lax.fori_loop(0, rows_pt // _L, zc, 0)

    e0 = pl.multiple_of(w * EPW, _L)
    g0 = pl.multiple_of(w * GPW, 8)
    pltpu.sync_copy(dst2_h.at[pl.ds(g0, GPW)], dv)

    def fuse_chunk(ch, _):
      ce = pl.multiple_of(e0 + ch * 8 * _L, _L)
      pltpu.sync_copy(src_h.at[pl.ds(ce, 8 * _L)], srcb)
      pltpu.sync_copy(kidx_h.at[pl.ds(ce, 8 * _L)], kb)
      def fuse(r, _):
        def fuse1(i, _):
          sl = pl.ds(r * _L + i * 16, 16)
          sv = srcb[sl]
          s7 = jnp.bitwise_and(sv, 7)
          u = s7 * K + kb[sl]
          gv[ch * 8 + r, pl.ds(i * 16, 16)] = (
              lax.shift_right_logical(u, 3) * nstride + (sv - s7) +
              jnp.bitwise_and(u, 7))
          return 0
        lax.fori_loop(0, 8, fuse1, 0)
        return 0
      lax.fori_loop(0, 8, fuse, 0)
      return 0
    lax.fori_loop(0, GPW // 8, fuse_chunk, 0)
    pltpu.sync_copy(gv, gidx2_h.at[pl.ds(g0, GPW)])
    plsc.subcore_barrier()

    def gather(j, b):
      pltpu.async_copy(tab_h.at[gv.at[j]], rows[b], sem[b])
    def gwait(b):
      pltpu.make_async_copy(tab_h.at[gv.at[0]], rows[b], sem[b]).wait()
    def scatter(j, b):
      pltpu.async_copy(rows[b], acc.at[dv.at[j]], sem[b], add=True)
    def swait(b):
      pltpu.make_async_copy(rows[b], acc.at[dv.at[0]], sem[b]).wait()

    for i in range(4):
      gather(i, i)
    for i in range(4):
      gather(i + 4, i + 4)
      gwait(i)
      scatter(i, i)
    def body8(o, _):
      jb = 4 + o * 8
      for i in range(8):
        j = jb + i
        bj = (4 + i) % 8
        bn = i
        swait(bn)
        gather(j + 4, bn)
        gwait(bj)
        scatter(j, bj)
      return 0
    lax.fori_loop(0, (GPW - 8) // 8, body8, 0)
    for i in range(4):
      gwait(4 + i)
      scatter(GPW - 4 + i, 4 + i)
    for b in range(8):
      swait(b)

    plsc.subcore_barrier()
    pltpu.sync_copy(acc.at[pl.ds(base, rows_pt)],
                    out_h.at[pl.ds(pl.multiple_of(c * NP + base, _L), rows_pt)])

  return convin


def _make_sc_counts(Ep, NP):
  GPW = Ep // (_L * _NW)
  rows_pt = NP // 16
  mesh = plsc.VectorSubcoreMesh(core_axis_name="c", subcore_axis_name="s")

  @functools.partial(
      pl.kernel,
      out_type=jax.ShapeDtypeStruct((2 * NP, _C), jnp.float32),
      mesh=mesh,
      compiler_params=pltpu.CompilerParams(use_tc_tiling_on_sc=False),
      scratch_types=[
          pltpu.VMEM((GPW, _L), jnp.int32),
          pltpu.VMEM((_L, _C), jnp.float32),
          pltpu.VMEM((_L, _C), jnp.float32),
          pltpu.VMEM_SHARED((NP, _C), jnp.float32),
          pltpu.SemaphoreType.DMA,
      ],
  )
  def counts(dst2_h, out_h, dv, ones, zb, acc, sem):
    c = lax.axis_index("c")
    s = lax.axis_index("s")
    w = s * 2 + c

    def zf(i, _):
      ones[i, :] = jnp.ones((_C,), jnp.float32)
      zb[i, :] = jnp.zeros((_C,), jnp.float32)
      return 0
    lax.fori_loop(0, _L, zf, 0)
    base = pl.multiple_of(s * rows_pt, _L)
    def zc(i, _):
      pltpu.sync_copy(zb, acc.at[pl.ds(pl.multiple_of(base + i * _L, _L), _L)])
      return 0
    lax.fori_loop(0, rows_pt // _L, zc, 0)
    g0 = pl.multiple_of(w * GPW, 8)
    pltpu.sync_copy(dst2_h.at[pl.ds(g0, GPW)], dv)
    plsc.subcore_barrier()

    def scat(j):
      pltpu.async_copy(ones, acc.at[dv.at[j]], sem, add=True)
    def swait():
      pltpu.make_async_copy(ones, acc.at[dv.at[0]], sem).wait()

    for j in range(8):
      scat(j)
    def body(o, _):
      for i in range(4):
        swait()
        scat(8 + o * 4 + i)
      return 0
    lax.fori_loop(0, (GPW - 8) // 4, body, 0)
    for _ in range(8):
      swait()

    plsc.subcore_barrier()
    pltpu.sync_copy(acc.at[pl.ds(base, rows_pt)],
                    out_h.at[pl.ds(pl.multiple_of(c * NP + base, _L), rows_pt)])

  return counts


def _tc_fused_table(xfn, ins, W2big, K, N8P):
  n8 = ins[0].shape[0]
  def body(*refs):
    in_refs = refs[:len(ins)]
    wr, orf, xs = refs[len(ins)], refs[len(ins) + 1], refs[len(ins) + 2]
    @pl.when(pl.program_id(0) == 0)
    def _():
      xs[pl.ds(0, n8), :] = xfn(*[r[...] for r in in_refs])
      if N8P > n8:
        xs[pl.ds(n8, N8P - n8), :] = jnp.zeros((N8P - n8, _L), jnp.float32)
    orf[0] = jnp.dot(xs[...], wr[...], preferred_element_type=jnp.float32)
  def full2d(a):
    return pl.BlockSpec(a.shape, lambda k3: (0, 0))
  return pl.pallas_call(
      body,
      grid=(K,),
      in_specs=[full2d(a) for a in ins] +
               [pl.BlockSpec((_L, _L), lambda k3: (0, k3))],
      out_specs=pl.BlockSpec((1, N8P, _L), lambda k3: (k3, 0, 0)),
      out_shape=jax.ShapeDtypeStruct((K, N8P, _L), jnp.float32),
      scratch_shapes=[pltpu.VMEM((N8P, _L), jnp.float32)],
  )(*ins, W2big)


def _tc_table(x128, W2big, K):
  n8 = x128.shape[0]
  def body(xr, wr, orf):
    orf[0] = jnp.dot(xr[...], wr[...], preferred_element_type=jnp.float32)
  return pl.pallas_call(
      body,
      grid=(K,),
      in_specs=[pl.BlockSpec((n8, _L), lambda k3: (0, 0)),
                pl.BlockSpec((_L, _L), lambda k3: (0, k3))],
      out_specs=pl.BlockSpec((1, n8, _L), lambda k3: (k3, 0, 0)),
      out_shape=jax.ShapeDtypeStruct((K, n8, _L), jnp.float32),
  )(x128, W2big)


def _bn_relu_expr(x, A, g, b, nf):
  s = jnp.dot(jnp.sum(x, axis=0, keepdims=True), A,
              preferred_element_type=jnp.float32) * (1.0 / nf)
  q = jnp.dot(jnp.sum(x * x, axis=0, keepdims=True), A,
              preferred_element_type=jnp.float32) * (1.0 / nf)
  v = q - s * s
  return jnp.maximum((x - s) / jnp.sqrt(v + 1e-3) * g + b, 0.0)


def _tc_bn(p0, p1, A, g, b, nf):
  def body(ar, br, Ar, gr, b2r, orf):
    orf[...] = _bn_relu_expr(ar[...] + br[...], Ar[...], gr[...], b2r[...], nf)
  return pl.pallas_call(
      body, out_shape=jax.ShapeDtypeStruct(p0.shape, jnp.float32),
  )(p0, p1, A, g, b)


def _tc_bn2(pb0, pb1, g3, b3, pc0, pc1, g4, b4, A, nf):
  def body(a0, a1, g3r, b3r, c0, c1, g4r, b4r, Ar, orf):
    orf[...] = (_bn_relu_expr(a0[...] + a1[...], Ar[...], g3r[...], b3r[...], nf) +
                _bn_relu_expr(c0[...] + c1[...], Ar[...], g4r[...], b4r[...], nf))
  return pl.pallas_call(
      body, out_shape=jax.ShapeDtypeStruct(pb0.shape, jnp.float32),
  )(pb0, pb1, g3, b3, pc0, pc1, g4, b4, A)


def _tc_div(s0, s1, c0, c1):
  def body(a, b, x, y, orf):
    orf[...] = (a[...] + b[...]) / jnp.maximum(x[...] + y[...], 1.0)
  return pl.pallas_call(
      body, out_shape=jax.ShapeDtypeStruct(s0.shape, jnp.float32),
  )(s0, s1, c0, c1)


def _tc_final(xnv1, ov, xpro):
  def body(a, b, c, orf):
    x = a[...] + b[...] - c[...]
    nrm = jnp.sqrt(jnp.sum(x * x))
    orf[...] = x / (nrm + 1e-12)
  return pl.pallas_call(
      body, out_shape=jax.ShapeDtypeStruct(xnv1.shape, jnp.float32),
  )(xnv1, ov, xpro)


def kernel(voxel_features, voxel_coords, edge_index, kernel_idx, batch_size,
           W_in, g1, b1, W_a, g2, b2, W_b, g3, b3, W_c, g4, b4):
  n = voxel_features.shape[0]
  E = edge_index.shape[1]
  K = W_in.shape[0]

  Ep = _ceil_to(E, _NW * _L * 40)
  NP = _ceil_to(n, 16 * _L)

  src = edge_index[0]
  dst = edge_index[1]
  srcp = jnp.pad(src, (0, Ep - E))
  kidxp = jnp.pad(kernel_idx, (0, Ep - E))
  dstp = jnp.pad(dst, (0, Ep - E), constant_values=n)
  dst2 = dstp.reshape(-1, _L)

  N8 = n * _C // _L
  N8P = _ceil_to(N8, 8)
  cpart = _make_sc_counts(Ep, NP)(dst2)

  def v128(t):
    return t[:n].reshape(-1, _L)

  c0 = v128(cpart)
  c1 = v128(cpart[NP:])
  nf = float(n)
  A = jnp.tile(jnp.eye(_C, dtype=jnp.float32), (8, 8))
  def t128(gv):
    return jnp.tile(gv, 8).reshape(1, _L)

  src64 = srcp.reshape(-1, 64)
  conv_big = _make_sc_gather_add(Ep, NP)
  conv_nm = _make_sc_gather_add(Ep, NP)

  def wbig(W):
    cin = W.shape[1]
    Wt = jnp.transpose(W, (1, 0, 2))
    Wt = jnp.pad(Wt, ((0, _C - cin), (0, 0), (0, 0)))
    big = (jnp.eye(8, dtype=jnp.float32)[:, None, :, None, None] *
           Wt[None, :, None, :, :])
    return big.reshape(_L, K * _L)

  def wtab(x128, W):
    x128p = jnp.pad(x128, ((0, N8P - N8), (0, 0)))
    return _tc_table(x128p, wbig(W), K).reshape(K * N8P * 8, _C)

  def bn_tab(p, g, b, W):
    tab = _tc_fused_table(
        lambda a, b_, Ar, gr, br: _bn_relu_expr(a + b_, Ar, gr, br, nf),
        [v128(p), v128(p[NP:]), A, t128(g), t128(b)], wbig(W), K, N8P)
    return tab.reshape(K * N8P * 8, _C)

  def nm_parts(x128):
    p = conv_nm(x128.reshape(n, _C), src64, dst64)
    return v128(p), v128(p[NP:])

  vf128 = jnp.pad(voxel_features,
                  ((0, 0), (0, _C - voxel_features.shape[1]))).reshape(-1, _L)
  gidx2, p = _make_sc_conv_in(Ep, NP, K, N8P * 8)(
      wtab(vf128, W_in), srcp, kidxp, dst2)
  gidx64 = gidx2.reshape(-1, 64)
  dst64 = dst2.reshape(-1, 64)
  pa = conv_big(bn_tab(p, g1, b1, W_a), gidx64, dst64)
  pb = conv_big(bn_tab(pa, g2, b2, W_b), gidx64, dst64)
  pc = conv_big(bn_tab(p, g1, b1, W_c), gidx64, dst64)
  x_pro = _tc_bn2(v128(pb), v128(pb[NP:]), t128(g3), t128(b3),
                  v128(pc), v128(pc[NP:]), t128(g4), t128(b4), A, nf)

  s0, s1 = nm_parts(x_pro)
  out_voxel = _tc_div(s0, s1, c0, c1)
  t0, t1 = nm_parts(out_voxel)
  ta2 = _tc_fused_table(
      lambda a, b_, x, y, o: (a + b_) / jnp.maximum(x + y, 1.0) - o,
      [t0, t1, c0, c1, out_voxel], wbig(W_a), K, N8P).reshape(K * N8P * 8, _C)
  q = conv_big(ta2, gidx64, dst64)
  r = conv_big(bn_tab(q, g2, b2, W_b), gidx64, dst64)
  xnv1 = _tc_bn(v128(r), v128(r[NP:]), A, t128(g3), t128(b3), nf)
  return _tc_final(xnv1, out_voxel, x_pro).reshape(n, _C)


# speedup vs baseline: 1.0043x; 1.0043x over previous
import functools

import jax
import jax.numpy as jnp
from jax import lax
from jax.experimental import pallas as pl
from jax.experimental.pallas import tpu as pltpu
from jax.experimental.pallas import tpu_sc as plsc

_L = 128
_NW = 32
_C = 16


def _ceil_to(x, m):
  return -(-x // m) * m


def _make_sc_gather_add(Ep, NP):
  GPW = Ep // (_L * _NW)
  rows_pt = NP // 16
  mesh = plsc.VectorSubcoreMesh(core_axis_name="c", subcore_axis_name="s")

  @functools.partial(
      pl.kernel,
      out_type=jax.ShapeDtypeStruct((2 * NP, _C), jnp.float32),
      mesh=mesh,
      compiler_params=pltpu.CompilerParams(use_tc_tiling_on_sc=False),
      scratch_types=[
          pltpu.VMEM((GPW, _L), jnp.int32),
          pltpu.VMEM((GPW, _L), jnp.int32),
          [pltpu.VMEM((_L, _C), jnp.float32) for _ in range(8)],
          pltpu.VMEM((_L, _C), jnp.float32),
          pltpu.VMEM_SHARED((NP, _C), jnp.float32),
          [pltpu.SemaphoreType.DMA for _ in range(8)],
      ],
  )
  def conv(tab_h, gidx2_h, dst2_h, out_h, gv, dv, rows, zb, acc, sem):
    c = lax.axis_index("c")
    s = lax.axis_index("s")
    w = s * 2 + c

    def zf(i, _):
      zb[i, :] = jnp.zeros((_C,), jnp.float32)
      return 0
    lax.fori_loop(0, _L, zf, 0)

    base = pl.multiple_of(s * rows_pt, _L)
    def zc(i, _):
      pltpu.sync_copy(zb, acc.at[pl.ds(pl.multiple_of(base + i * _L, _L), _L)])
      return 0
    lax.fori_loop(0, rows_pt // _L, zc, 0)

    g0 = pl.multiple_of(w * GPW, 8)
    pltpu.sync_copy(gidx2_h.at[pl.ds(g0, GPW)], gv)
    pltpu.sync_copy(dst2_h.at[pl.ds(g0, GPW)], dv)
    plsc.subcore_barrier()

    def gather(j, b):
      pltpu.async_copy(tab_h.at[gv.at[j]], rows[b], sem[b])

    def gwait(b):
      pltpu.make_async_copy(tab_h.at[gv.at[0]], rows[b], sem[b]).wait()

    def scatter(j, b):
      pltpu.async_copy(rows[b], acc.at[dv.at[j]], sem[b], add=True)

    def swait(b):
      pltpu.make_async_copy(rows[b], acc.at[dv.at[0]], sem[b]).wait()

    for i in range(4):
      gather(i, i)
    for i in range(4):
      gather(i + 4, i + 4)
      gwait(i)
      scatter(i, i)

    def body8(o, _):
      jb = 4 + o * 8
      for i in range(8):
        j = jb + i
        bj = (4 + i) % 8
        bn = i
        swait(bn)
        gather(j + 4, bn)
        gwait(bj)
        scatter(j, bj)
      return 0
    lax.fori_loop(0, (GPW - 8) // 8, body8, 0)

    for i in range(4):
      gwait(4 + i)
      scatter(GPW - 4 + i, 4 + i)
    for b in range(8):
      swait(b)

    plsc.subcore_barrier()
    pltpu.sync_copy(acc.at[pl.ds(base, rows_pt)],
                    out_h.at[pl.ds(pl.multiple_of(c * NP + base, _L), rows_pt)])

  return conv


def _make_sc_conv_in(Ep, NP, K, nstride):
  GPW = Ep // (_L * _NW)
  EPW = Ep // _NW
  rows_pt = NP // 16
  mesh = plsc.VectorSubcoreMesh(core_axis_name="c", subcore_axis_name="s")

  @functools.partial(
      pl.kernel,
      out_type=(jax.ShapeDtypeStruct((Ep // _L, _L), jnp.int32),
                jax.ShapeDtypeStruct((2 * NP, _C), jnp.float32)),
      mesh=mesh,
      compiler_params=pltpu.CompilerParams(use_tc_tiling_on_sc=False),
      scratch_types=[
          pltpu.VMEM((8 * _L,), jnp.int32),
          pltpu.VMEM((8 * _L,), jnp.int32),
          pltpu.VMEM((GPW, _L), jnp.int32),
          pltpu.VMEM((GPW, _L), jnp.int32),
          [pltpu.VMEM((_L, _C), jnp.float32) for _ in range(8)],
          pltpu.VMEM((_L, _C), jnp.float32),
          pltpu.VMEM_SHARED((NP, _C), jnp.float32),
          [pltpu.SemaphoreType.DMA for _ in range(8)],
      ],
  )
  def convin(tab_h, src_h, kidx_h, dst2_h, gidx2_h, out_h, srcb, kb, gv, dv,
             rows, zb, acc, sem):
    c = lax.axis_index("c")
    s = lax.axis_index("s")
    w = s * 2 + c

    def zf(i, _):
      zb[i, :] = jnp.zeros((_C,), jnp.float32)
      return 0
    lax.fori_loop(0, _L, zf, 0)
    base = pl.multiple_of(s * rows_pt, _L)
    def zc(i, _):
      pltpu.sync_copy(zb, acc.at[pl.ds(pl.multiple_of(base + i * _L, _L), _L)])
      return 0
    lax.fori_loop(0, rows_pt // _L, zc, 0)

    e0 = pl.multiple_of(w * EPW, _L)
    g0 = pl.multiple_of(w * GPW, 8)
    pltpu.sync_copy(dst2_h.at[pl.ds(g0, GPW)], dv)

    def fuse_chunk(ch, _):
      ce = pl.multiple_of(e0 + ch * 8 * _L, _L)
      pltpu.sync_copy(src_h.at[pl.ds(ce, 8 * _L)], srcb)
      pltpu.sync_copy(kidx_h.at[pl.ds(ce, 8 * _L)], kb)
      def fuse(r, _):
        def fuse1(i, _):
          sl = pl.ds(r * _L + i * 16, 16)
          sv = srcb[sl]
          s7 = jnp.bitwise_and(sv, 7)
          u = s7 * K + kb[sl]
          gv[ch * 8 + r, pl.ds(i * 16, 16)] = (
              lax.shift_right_logical(u, 3) * nstride + (sv - s7) +
              jnp.bitwise_and(u, 7))
          return 0
        lax.fori_loop(0, 8, fuse1, 0)
        return 0
      lax.fori_loop(0, 8, fuse, 0)
      return 0
    lax.fori_loop(0, GPW // 8, fuse_chunk, 0)
    pltpu.sync_copy(gv, gidx2_h.at[pl.ds(g0, GPW)])
    plsc.subcore_barrier()

    def gather(j, b):
      pltpu.async_copy(tab_h.at[gv.at[j]], rows[b], sem[b])
    def gwait(b):
      pltpu.make_async_copy(tab_h.at[gv.at[0]], rows[b], sem[b]).wait()
    def scatter(j, b):
      pltpu.async_copy(rows[b], acc.at[dv.at[j]], sem[b], add=True)
    def swait(b):
      pltpu.make_async_copy(rows[b], acc.at[dv.at[0]], sem[b]).wait()

    for i in range(4):
      gather(i, i)
    for i in range(4):
      gather(i + 4, i + 4)
      gwait(i)
      scatter(i, i)
    def body8(o, _):
      jb = 4 + o * 8
      for i in range(8):
        j = jb + i
        bj = (4 + i) % 8
        bn = i
        swait(bn)
        gather(j + 4, bn)
        gwait(bj)
        scatter(j, bj)
      return 0
    lax.fori_loop(0, (GPW - 8) // 8, body8, 0)
    for i in range(4):
      gwait(4 + i)
      scatter(GPW - 4 + i, 4 + i)
    for b in range(8):
      swait(b)

    plsc.subcore_barrier()
    pltpu.sync_copy(acc.at[pl.ds(base, rows_pt)],
                    out_h.at[pl.ds(pl.multiple_of(c * NP + base, _L), rows_pt)])

  return convin


def _make_sc_counts(Ep, NP):
  GPW = Ep // (_L * _NW)
  rows_pt = NP // 16
  mesh = plsc.VectorSubcoreMesh(core_axis_name="c", subcore_axis_name="s")

  @functools.partial(
      pl.kernel,
      out_type=jax.ShapeDtypeStruct((2 * NP, _C), jnp.float32),
      mesh=mesh,
      compiler_params=pltpu.CompilerParams(use_tc_tiling_on_sc=False),
      scratch_types=[
          pltpu.VMEM((GPW, _L), jnp.int32),
          pltpu.VMEM((_L, _C), jnp.float32),
          pltpu.VMEM((_L, _C), jnp.float32),
          pltpu.VMEM_SHARED((NP, _C), jnp.float32),
          pltpu.SemaphoreType.DMA,
      ],
  )
  def counts(dst2_h, out_h, dv, ones, zb, acc, sem):
    c = lax.axis_index("c")
    s = lax.axis_index("s")
    w = s * 2 + c

    def zf(i, _):
      ones[i, :] = jnp.ones((_C,), jnp.float32)
      zb[i, :] = jnp.zeros((_C,), jnp.float32)
      return 0
    lax.fori_loop(0, _L, zf, 0)
    base = pl.multiple_of(s * rows_pt, _L)
    def zc(i, _):
      pltpu.sync_copy(zb, acc.at[pl.ds(pl.multiple_of(base + i * _L, _L), _L)])
      return 0
    lax.fori_loop(0, rows_pt // _L, zc, 0)
    g0 = pl.multiple_of(w * GPW, 8)
    pltpu.sync_copy(dst2_h.at[pl.ds(g0, GPW)], dv)
    plsc.subcore_barrier()

    def scat(j):
      pltpu.async_copy(ones, acc.at[dv.at[j]], sem, add=True)
    def swait():
      pltpu.make_async_copy(ones, acc.at[dv.at[0]], sem).wait()

    for j in range(8):
      scat(j)
    def body(o, _):
      for i in range(4):
        swait()
        scat(8 + o * 4 + i)
      return 0
    lax.fori_loop(0, (GPW - 8) // 4, body, 0)
    for _ in range(8):
      swait()

    plsc.subcore_barrier()
    pltpu.sync_copy(acc.at[pl.ds(base, rows_pt)],
                    out_h.at[pl.ds(pl.multiple_of(c * NP + base, _L), rows_pt)])

  return counts


def _tc_fused_table(xfn, ins, W2big, K, N8P):
  n8 = ins[0].shape[0]
  def body(*refs):
    in_refs = refs[:len(ins)]
    wr, orf, xs = refs[len(ins)], refs[len(ins) + 1], refs[len(ins) + 2]
    @pl.when(pl.program_id(0) == 0)
    def _():
      xs[pl.ds(0, n8), :] = xfn(*[r[...] for r in in_refs])
      if N8P > n8:
        xs[pl.ds(n8, N8P - n8), :] = jnp.zeros((N8P - n8, _L), jnp.float32)
    orf[0] = jnp.dot(xs[...], wr[...], preferred_element_type=jnp.float32)
  def full2d(a):
    return pl.BlockSpec(a.shape, lambda k3: (0, 0))
  return pl.pallas_call(
      body,
      grid=(K,),
      in_specs=[full2d(a) for a in ins] +
               [pl.BlockSpec((_L, _L), lambda k3: (0, k3))],
      out_specs=pl.BlockSpec((1, N8P, _L), lambda k3: (k3, 0, 0)),
      out_shape=jax.ShapeDtypeStruct((K, N8P, _L), jnp.float32),
      scratch_shapes=[pltpu.VMEM((N8P, _L), jnp.float32)],
  )(*ins, W2big)


def _tc_table(x128, W2big, K):
  n8 = x128.shape[0]
  def body(xr, wr, orf):
    orf[0] = jnp.dot(xr[...], wr[...], preferred_element_type=jnp.float32)
  return pl.pallas_call(
      body,
      grid=(K,),
      in_specs=[pl.BlockSpec((n8, _L), lambda k3: (0, 0)),
                pl.BlockSpec((_L, _L), lambda k3: (0, k3))],
      out_specs=pl.BlockSpec((1, n8, _L), lambda k3: (k3, 0, 0)),
      out_shape=jax.ShapeDtypeStruct((K, n8, _L), jnp.float32),
  )(x128, W2big)


def _bn_relu_expr(x, A, g, b, nf):
  s = jnp.dot(jnp.sum(x, axis=0, keepdims=True), A,
              preferred_element_type=jnp.float32) * (1.0 / nf)
  q = jnp.dot(jnp.sum(x * x, axis=0, keepdims=True), A,
              preferred_element_type=jnp.float32) * (1.0 / nf)
  v = q - s * s
  return jnp.maximum((x - s) / jnp.sqrt(v + 1e-3) * g + b, 0.0)


def _tc_bn(p0, p1, A, g, b, nf):
  def body(ar, br, Ar, gr, b2r, orf):
    orf[...] = _bn_relu_expr(ar[...] + br[...], Ar[...], gr[...], b2r[...], nf)
  return pl.pallas_call(
      body, out_shape=jax.ShapeDtypeStruct(p0.shape, jnp.float32),
  )(p0, p1, A, g, b)


def _tc_bn2(pb0, pb1, g3, b3, pc0, pc1, g4, b4, A, nf):
  def body(a0, a1, g3r, b3r, c0, c1, g4r, b4r, Ar, orf):
    orf[...] = (_bn_relu_expr(a0[...] + a1[...], Ar[...], g3r[...], b3r[...], nf) +
                _bn_relu_expr(c0[...] + c1[...], Ar[...], g4r[...], b4r[...], nf))
  return pl.pallas_call(
      body, out_shape=jax.ShapeDtypeStruct(pb0.shape, jnp.float32),
  )(pb0, pb1, g3, b3, pc0, pc1, g4, b4, A)


def _tc_div(s0, s1, c0, c1):
  def body(a, b, x, y, orf):
    orf[...] = (a[...] + b[...]) / jnp.maximum(x[...] + y[...], 1.0)
  return pl.pallas_call(
      body, out_shape=jax.ShapeDtypeStruct(s0.shape, jnp.float32),
  )(s0, s1, c0, c1)


def _tc_final(xnv1, ov, xpro):
  def body(a, b, c, orf):
    x = a[...] + b[...] - c[...]
    nrm = jnp.sqrt(jnp.sum(x * x))
    orf[...] = x / (nrm + 1e-12)
  return pl.pallas_call(
      body, out_shape=jax.ShapeDtypeStruct(xnv1.shape, jnp.float32),
  )(xnv1, ov, xpro)


def kernel(voxel_features, voxel_coords, edge_index, kernel_idx, batch_size,
           W_in, g1, b1, W_a, g2, b2, W_b, g3, b3, W_c, g4, b4):
  n = voxel_features.shape[0]
  E = edge_index.shape[1]
  K = W_in.shape[0]

  Ep = _ceil_to(E, _NW * _L * 40)
  NP = _ceil_to(n, 16 * _L)

  src = edge_index[0]
  dst = edge_index[1]
  srcp = jnp.pad(src, (0, Ep - E))
  kidxp = jnp.pad(kernel_idx, (0, Ep - E))
  dstp = jnp.pad(dst, (0, Ep - E), constant_values=n)
  dst2 = dstp.reshape(-1, _L)

  N8 = n * _C // _L
  N8P = _ceil_to(N8, 8)
  cpart = _make_sc_counts(Ep, NP)(dst2)

  def v128(t):
    return t[:n].reshape(-1, _L)

  c0 = v128(cpart)
  c1 = v128(cpart[NP:])
  nf = float(n)
  A = jnp.tile(jnp.eye(_C, dtype=jnp.float32), (8, 8))
  def t128(gv):
    return jnp.tile(gv, 8).reshape(1, _L)

  src2 = srcp.reshape(-1, _L)
  conv_big = _make_sc_gather_add(Ep, NP)
  conv_nm = _make_sc_gather_add(Ep, NP)

  def wbig(W):
    cin = W.shape[1]
    Wt = jnp.transpose(W, (1, 0, 2))
    Wt = jnp.pad(Wt, ((0, _C - cin), (0, 0), (0, 0)))
    big = (jnp.eye(8, dtype=jnp.float32)[:, None, :, None, None] *
           Wt[None, :, None, :, :])
    return big.reshape(_L, K * _L)

  def wtab(x128, W):
    x128p = jnp.pad(x128, ((0, N8P - N8), (0, 0)))
    return _tc_table(x128p, wbig(W), K).reshape(K * N8P * 8, _C)

  def bn_tab(p, g, b, W):
    tab = _tc_fused_table(
        lambda a, b_, Ar, gr, br: _bn_relu_expr(a + b_, Ar, gr, br, nf),
        [v128(p), v128(p[NP:]), A, t128(g), t128(b)], wbig(W), K, N8P)
    return tab.reshape(K * N8P * 8, _C)

  def nm_parts(x128):
    p = conv_nm(x128.reshape(n, _C), src2, dst2)
    return v128(p), v128(p[NP:])

  vf128 = jnp.pad(voxel_features,
                  ((0, 0), (0, _C - voxel_features.shape[1]))).reshape(-1, _L)
  gidx2, p = _make_sc_conv_in(Ep, NP, K, N8P * 8)(
      wtab(vf128, W_in), srcp, kidxp, dst2)
  pa = conv_big(bn_tab(p, g1, b1, W_a), gidx2, dst2)
  pb = conv_big(bn_tab(pa, g2, b2, W_b), gidx2, dst2)
  pc = conv_big(bn_tab(p, g1, b1, W_c), gidx2, dst2)
  x_pro = _tc_bn2(v128(pb), v128(pb[NP:]), t128(g3), t128(b3),
                  v128(pc), v128(pc[NP:]), t128(g4), t128(b4), A, nf)

  s0, s1 = nm_parts(x_pro)
  out_voxel = _tc_div(s0, s1, c0, c1)
  t0, t1 = nm_parts(out_voxel)
  ta2 = _tc_fused_table(
      lambda a, b_, x, y, o: (a + b_) / jnp.maximum(x + y, 1.0) - o,
      [t0, t1, c0, c1, out_voxel], wbig(W_a), K, N8P).reshape(K * N8P * 8, _C)
  q = conv_big(ta2, gidx2, dst2)
  r = conv_big(bn_tab(q, g2, b2, W_b), gidx2, dst2)
  xnv1 = _tc_bn(v128(r), v128(r[NP:]), A, t128(g3), t128(b3), nf)
  return _tc_final(xnv1, out_voxel, x_pro).reshape(n, _C)
